# baseline (speedup 1.0000x reference)
import functools

import jax
import jax.numpy as jnp
from jax import lax
from jax.experimental import pallas as pl
from jax.experimental.pallas import tpu as pltpu
from jax.experimental.pallas import tpu_sc as plsc

CONF = 0.01
TOPK = 200
NMS_T = 0.45
N = 8732
NP = 8736
NC = 21
NCP = 40
CAP = 208
NV = CAP // 16
CAPX = 256
NBLK = NP // 16
OUTF = 1024


def _pre_body(loc_ref, conf_ref, dbox_ref, sc_ref, bx_ref, th_ref):
    ct = conf_ref[0].T
    m = jnp.max(ct, axis=0, keepdims=True)
    e = jnp.exp(ct - m)
    z = e[0:1]
    for j in range(1, NC):
        z = z + e[j:j + 1]
    p = e / z
    st = jnp.where(p > CONF, p, 0.0)
    sc_ref[0] = jnp.concatenate([st, jnp.zeros((NC, NP - N), jnp.float32)], axis=1)

    loc = loc_ref[0]
    dbox = dbox_ref[...]
    cxcy = dbox[:, :2] + loc[:, :2] * 0.1 * dbox[:, :2]
    wh = dbox[:, 2:] * jnp.exp(loc[:, 2:] * 0.2)
    xy1 = cxcy - wh / 2.0
    xy2 = xy1 + wh
    bt = jnp.concatenate([xy1, xy2], axis=1).T
    bx_ref[0] = jnp.concatenate([bt, jnp.zeros((4, NP - N), jnp.float32)], axis=1)

    bits = lax.bitcast_convert_type(sc_ref[0], jnp.int32)

    def body(_, carry):
        lo, hi = carry
        mid = (lo + hi) // 2
        cnt = jnp.sum((bits > mid).astype(jnp.float32), axis=1, keepdims=True)
        pred = cnt >= float(TOPK)
        return jnp.where(pred, mid + 1, lo), jnp.where(pred, hi, mid)

    LOB = 0x3C23D70A
    lo0 = jnp.full((NC, 1), LOB, jnp.int32)
    hi0 = jnp.full((NC, 1), 0x3F800000, jnp.int32)
    _, hi = lax.fori_loop(0, 26, body, (lo0, hi0))
    thf = jnp.where(hi == LOB, 0.0, lax.bitcast_convert_type(hi, jnp.float32))
    th_ref[0] = jnp.concatenate(
        [thf.T, jnp.full((1, NCP - NC), 2.0, jnp.float32)], axis=1)


def _preprocess(loc_data, conf_data, dbox_list):
    B = loc_data.shape[0]
    return pl.pallas_call(
        _pre_body,
        grid=(B,),
        in_specs=[
            pl.BlockSpec((1, N, 4), lambda b: (b, 0, 0)),
            pl.BlockSpec((1, N, NC), lambda b: (b, 0, 0)),
            pl.BlockSpec((N, 4), lambda b: (0, 0)),
        ],
        out_specs=[
            pl.BlockSpec((1, NC, NP), lambda b: (b, 0, 0)),
            pl.BlockSpec((1, 4, NP), lambda b: (b, 0, 0)),
            pl.BlockSpec((1, 1, NCP), lambda b: (b, 0, 0)),
        ],
        out_shape=[
            jax.ShapeDtypeStruct((B, NC, NP), jnp.float32),
            jax.ShapeDtypeStruct((B, 4, NP), jnp.float32),
            jax.ShapeDtypeStruct((B, 1, NCP), jnp.float32),
        ],
        compiler_params=pltpu.CompilerParams(
            dimension_semantics=("parallel",)),
    )(loc_data, conf_data, dbox_list)


def _nms_body(sc_hbm, bx_hbm, th_hbm, out_hbm,
              bxp0, bxp1, bxp2, bxp3, srow, thv,
              ss, idxs, x1s, y1s, x2s, y2s, areas, supp, pvm, outflat):
    wid = lax.axis_index("s") * 2 + lax.axis_index("c")
    b = wid // 2
    half = wid % 2

    pltpu.sync_copy(bx_hbm.at[b, 0], bxp0)
    pltpu.sync_copy(bx_hbm.at[b, 1], bxp1)
    pltpu.sync_copy(bx_hbm.at[b, 2], bxp2)
    pltpu.sync_copy(bx_hbm.at[b, 3], bxp3)
    pltpu.sync_copy(th_hbm.at[b, 0], thv)

    lane = lax.iota(jnp.int32, 16)
    zero16 = jnp.zeros((16,), jnp.float32)
    row_mask = lane < 5

    def do_class(c, th):
        pltpu.sync_copy(sc_hbm.at[b, c], srow)

        for j in range(NV):
            ss[pl.ds(j * 16, 16)] = jnp.full((16,), -1.0, jnp.float32)
            idxs[pl.ds(j * 16, 16)] = jnp.zeros((16,), jnp.int32)

        def comp_blk(h, cnt):
            for u in range(2):
                base = (h * 2 + u) * 16
                v = srow[pl.ds(base, 16)]
                msk = (v >= th) & (v > 0.0)
                woff = jnp.minimum(cnt, CAP - 16)
                plsc.store_compressed(ss.at[pl.ds(woff, 16)], v, mask=msk)
                plsc.store_compressed(idxs.at[pl.ds(woff, 16)],
                                      base + lane, mask=msk)
                cnt = cnt + plsc.all_reduce_population_count(msk)[0]
            return cnt

        cnt = lax.fori_loop(0, NBLK // 2, comp_blk, jnp.int32(0))
        cnt = jnp.minimum(cnt, CAP)
        ss[pl.ds(cnt, 16)] = jnp.full((16,), -1.0, jnp.float32)
        idxs[pl.ds(cnt, 16)] = jnp.zeros((16,), jnp.int32)

        pvmv = jnp.full((16,), -1.0, jnp.float32)
        for j in range(NV):
            base = j * 16
            iv = idxs[pl.ds(base, 16)]
            x1v = plsc.load_gather(bxp0, [iv])
            y1v = plsc.load_gather(bxp1, [iv])
            x2v = plsc.load_gather(bxp2, [iv])
            y2v = plsc.load_gather(bxp3, [iv])
            x1s[pl.ds(base, 16)] = x1v
            y1s[pl.ds(base, 16)] = y1v
            x2s[pl.ds(base, 16)] = x2v
            y2s[pl.ds(base, 16)] = y2v
            areas[pl.ds(base, 16)] = (x2v - x1v) * (y2v - y1v)
            supp[pl.ds(base, 16)] = zero16
            pvmv = jnp.where(lane == j, jnp.max(ss[pl.ds(base, 16)]), pvmv)
        pvm[...] = pvmv

        def extract(k, carry):
            pv = pvm[...]
            gm = jnp.max(pv)
            v0 = jnp.minimum(plsc.all_reduce_ffs(pv == gm)[0], jnp.int32(15))
            base = v0 * 16
            sv = ss[pl.ds(base, 16)]
            l = jnp.minimum(plsc.all_reduce_ffs(sv == gm)[0], jnp.int32(15))
            sv2 = jnp.where(lane == l, -1.0, sv)
            ss[pl.ds(base, 16)] = sv2
            pvm[...] = jnp.where(lane == v0, jnp.max(sv2), pv)

            slot = base + l
            sup = supp[pl.ds(slot, 16)][0]
            bx1 = x1s[pl.ds(slot, 16)][0]
            by1 = y1s[pl.ds(slot, 16)][0]
            bx2 = x2s[pl.ds(slot, 16)][0]
            by2 = y2s[pl.ds(slot, 16)][0]
            barea = areas[pl.ds(slot, 16)][0]
            kept = (gm > 0.0) & (sup == 0.0)
            keptf = jnp.where(kept, 1.0, 0.0)

            rv = jnp.where(lane == 0, gm, zero16)
            rv = jnp.where(lane == 1, bx1, rv)
            rv = jnp.where(lane == 2, by1, rv)
            rv = jnp.where(lane == 3, bx2, rv)
            rv = jnp.where(lane == 4, by2, rv)
            plsc.store_scatter(outflat, [k * 5 + lane], rv * keptf,
                               mask=row_mask)

            @pl.when(kept)
            def _():
                one16 = jnp.full((16,), 1.0, jnp.float32)
                for j in range(NV):
                    sbase = j * 16
                    x1v = x1s[pl.ds(sbase, 16)]
                    y1v = y1s[pl.ds(sbase, 16)]
                    x2v = x2s[pl.ds(sbase, 16)]
                    y2v = y2s[pl.ds(sbase, 16)]
                    av = areas[pl.ds(sbase, 16)]
                    iw = jnp.maximum(jnp.minimum(bx2, x2v) - jnp.maximum(bx1, x1v), 0.0)
                    ih = jnp.maximum(jnp.minimum(by2, y2v) - jnp.maximum(by1, y1v), 0.0)
                    inter = iw * ih
                    iou = inter / (barea + av - inter + 1e-12)
                    plsc.store_scatter(supp, [sbase + lane], one16,
                                       mask=iou > NMS_T)
            return carry

        lax.fori_loop(0, TOPK, extract, jnp.int32(0))
        pltpu.sync_copy(outflat.at[pl.ds(0, TOPK * 5)], out_hbm.at[b, c])

    @pl.when(half == 0)
    def _():
        def zblk(j, _):
            outflat[pl.ds(j * 16, 16)] = zero16
            return _
        lax.fori_loop(0, OUTF // 16, zblk, jnp.int32(0))
        pltpu.sync_copy(outflat.at[pl.ds(0, TOPK * 5)], out_hbm.at[b, 0])

    first = 1 + half * 10

    def cls_loop(i, _):
        th = thv[pl.ds(first + i, 16)][0]
        do_class(first + i, th)
        return _

    lax.fori_loop(0, 10, cls_loop, jnp.int32(0))


def _nms_sc(sc, bx, th, B):
    mesh = plsc.VectorSubcoreMesh(core_axis_name="c", subcore_axis_name="s")
    kern = functools.partial(
        pl.kernel,
        mesh=mesh,
        out_type=jax.ShapeDtypeStruct((B, NC, TOPK * 5), jnp.float32),
        compiler_params=pltpu.CompilerParams(
            needs_layout_passes=False, use_tc_tiling_on_sc=False),
        scratch_types=[
            pltpu.VMEM((NP,), jnp.float32),
            pltpu.VMEM((NP,), jnp.float32),
            pltpu.VMEM((NP,), jnp.float32),
            pltpu.VMEM((NP,), jnp.float32),
            pltpu.VMEM((NP,), jnp.float32),
            pltpu.VMEM((NCP,), jnp.float32),
            pltpu.VMEM((CAPX,), jnp.float32),
            pltpu.VMEM((CAPX,), jnp.int32),
            pltpu.VMEM((CAPX,), jnp.float32),
            pltpu.VMEM((CAPX,), jnp.float32),
            pltpu.VMEM((CAPX,), jnp.float32),
            pltpu.VMEM((CAPX,), jnp.float32),
            pltpu.VMEM((CAPX,), jnp.float32),
            pltpu.VMEM((CAPX,), jnp.float32),
            pltpu.VMEM((16,), jnp.float32),
            pltpu.VMEM((OUTF,), jnp.float32),
        ],
    )(_nms_body)
    return kern(sc, bx, th)


def kernel(loc_data, conf_data, dbox_list):
    B = loc_data.shape[0]
    sc, bx, th = _preprocess(loc_data, conf_data, dbox_list)
    out = _nms_sc(sc, bx, th, B)
    return out.reshape(B, NC, TOPK, 5)


# speedup vs baseline: 1.4486x; 1.4486x over previous
import functools

import jax
import jax.numpy as jnp
from jax import lax
from jax.experimental import pallas as pl
from jax.experimental.pallas import tpu as pltpu
from jax.experimental.pallas import tpu_sc as plsc

CONF = 0.01
TOPK = 200
NMS_T = 0.45
N = 8732
NP = 8736
NC = 21
NCP = 40
CAP = 208
NV = CAP // 16
CAPW = 224
CAPX = 256
NBLK = NP // 16
OUTF = 1024


def _pre_body(loc_ref, conf_ref, dbox_ref, sc_ref, bx_ref, th_ref):
    ct = conf_ref[0].T
    m = jnp.max(ct, axis=0, keepdims=True)
    e = jnp.exp(ct - m)
    z = e[0:1]
    for j in range(1, NC):
        z = z + e[j:j + 1]
    p = e / z
    st = jnp.where(p > CONF, p, 0.0)
    sc_ref[0] = jnp.concatenate([st, jnp.zeros((NC, NP - N), jnp.float32)], axis=1)

    loc = loc_ref[0]
    dbox = dbox_ref[...]
    cxcy = dbox[:, :2] + loc[:, :2] * 0.1 * dbox[:, :2]
    wh = dbox[:, 2:] * jnp.exp(loc[:, 2:] * 0.2)
    xy1 = cxcy - wh / 2.0
    xy2 = xy1 + wh
    bt = jnp.concatenate([xy1, xy2], axis=1).T
    bx_ref[0] = jnp.concatenate([bt, jnp.zeros((4, NP - N), jnp.float32)], axis=1)

    bits = lax.bitcast_convert_type(sc_ref[0], jnp.int32)

    def body(_, carry):
        lo, hi = carry
        mid = (lo + hi) // 2
        cnt = jnp.sum((bits > mid).astype(jnp.float32), axis=1, keepdims=True)
        pred = cnt >= float(TOPK)
        return jnp.where(pred, mid + 1, lo), jnp.where(pred, hi, mid)

    LOB = 0x3C23D70A
    lo0 = jnp.full((NC, 1), LOB, jnp.int32)
    hi0 = jnp.full((NC, 1), 0x3F800000, jnp.int32)
    _, hi = lax.fori_loop(0, 26, body, (lo0, hi0))
    thf = jnp.where(hi == LOB, 0.0, lax.bitcast_convert_type(hi, jnp.float32))
    th_ref[0] = jnp.concatenate(
        [thf.T, jnp.full((1, NCP - NC), 2.0, jnp.float32)], axis=1)


def _preprocess(loc_data, conf_data, dbox_list):
    B = loc_data.shape[0]
    return pl.pallas_call(
        _pre_body,
        grid=(B,),
        in_specs=[
            pl.BlockSpec((1, N, 4), lambda b: (b, 0, 0)),
            pl.BlockSpec((1, N, NC), lambda b: (b, 0, 0)),
            pl.BlockSpec((N, 4), lambda b: (0, 0)),
        ],
        out_specs=[
            pl.BlockSpec((1, NC, NP), lambda b: (b, 0, 0)),
            pl.BlockSpec((1, 4, NP), lambda b: (b, 0, 0)),
            pl.BlockSpec((1, 1, NCP), lambda b: (b, 0, 0)),
        ],
        out_shape=[
            jax.ShapeDtypeStruct((B, NC, NP), jnp.float32),
            jax.ShapeDtypeStruct((B, 4, NP), jnp.float32),
            jax.ShapeDtypeStruct((B, 1, NCP), jnp.float32),
        ],
        compiler_params=pltpu.CompilerParams(
            dimension_semantics=("parallel",)),
    )(loc_data, conf_data, dbox_list)


def _nms_body(sc_hbm, bx_hbm, th_hbm, out_hbm,
              bxp0, bxp1, bxp2, bxp3, srow, thv,
              ss, idxs, x1s, y1s, x2s, y2s, areas, supp, pvm, outflat):
    wid = lax.axis_index("s") * 2 + lax.axis_index("c")
    b = wid // 2
    half = wid % 2

    pltpu.sync_copy(bx_hbm.at[b, 0], bxp0)
    pltpu.sync_copy(bx_hbm.at[b, 1], bxp1)
    pltpu.sync_copy(bx_hbm.at[b, 2], bxp2)
    pltpu.sync_copy(bx_hbm.at[b, 3], bxp3)
    pltpu.sync_copy(th_hbm.at[b, 0], thv)

    lane = lax.iota(jnp.int32, 16)
    zero16 = jnp.zeros((16,), jnp.float32)
    row_mask = lane < 5

    def do_class(c, th):
        pltpu.sync_copy(sc_hbm.at[b, c], srow)

        for j in range(NV + 1):
            ss[pl.ds(j * 16, 16)] = jnp.full((16,), -1.0, jnp.float32)
            idxs[pl.ds(j * 16, 16)] = jnp.zeros((16,), jnp.int32)

        def comp_blk(h, cnt):
            for u in range(2):
                base = (h * 2 + u) * 16
                v = srow[pl.ds(base, 16)]
                msk = (v >= th) & (v > 0.0)
                woff = jnp.minimum(cnt, CAPW - 16)
                plsc.store_compressed(ss.at[pl.ds(woff, 16)], v, mask=msk)
                plsc.store_compressed(idxs.at[pl.ds(woff, 16)],
                                      base + lane, mask=msk)
                cnt = cnt + plsc.all_reduce_population_count(msk)[0]
            return cnt

        cnt = lax.fori_loop(0, NBLK // 2, comp_blk, jnp.int32(0))
        cnt = jnp.minimum(cnt, CAP)
        ss[pl.ds(cnt, 16)] = jnp.full((16,), -1.0, jnp.float32)
        idxs[pl.ds(cnt, 16)] = jnp.zeros((16,), jnp.int32)

        pvmv = jnp.full((16,), -1.0, jnp.float32)
        for j in range(NV):
            base = j * 16
            iv = idxs[pl.ds(base, 16)]
            x1v = plsc.load_gather(bxp0, [iv])
            y1v = plsc.load_gather(bxp1, [iv])
            x2v = plsc.load_gather(bxp2, [iv])
            y2v = plsc.load_gather(bxp3, [iv])
            x1s[pl.ds(base, 16)] = x1v
            y1s[pl.ds(base, 16)] = y1v
            x2s[pl.ds(base, 16)] = x2v
            y2s[pl.ds(base, 16)] = y2v
            areas[pl.ds(base, 16)] = (x2v - x1v) * (y2v - y1v)
            supp[pl.ds(base, 16)] = zero16
            pvmv = jnp.where(lane == j, jnp.max(ss[pl.ds(base, 16)]), pvmv)
        pvm[...] = pvmv

        def extract(k, carry):
            pv = pvm[...]
            gm = jnp.max(pv)
            v0 = jnp.minimum(plsc.all_reduce_ffs(pv == gm)[0], jnp.int32(15))
            base = v0 * 16
            sv = ss[pl.ds(base, 16)]
            l = jnp.minimum(plsc.all_reduce_ffs(sv == gm)[0], jnp.int32(15))
            sv2 = jnp.where(lane == l, -1.0, sv)
            ss[pl.ds(base, 16)] = sv2
            pvm[...] = jnp.where(lane == v0, jnp.max(sv2), pv)

            slot = base + l
            sup = supp[pl.ds(slot, 16)][0]
            bx1 = x1s[pl.ds(slot, 16)][0]
            by1 = y1s[pl.ds(slot, 16)][0]
            bx2 = x2s[pl.ds(slot, 16)][0]
            by2 = y2s[pl.ds(slot, 16)][0]
            barea = areas[pl.ds(slot, 16)][0]
            kept = (gm > 0.0) & (sup == 0.0)
            keptf = jnp.where(kept, 1.0, 0.0)

            rv = jnp.where(lane == 0, gm, zero16)
            rv = jnp.where(lane == 1, bx1, rv)
            rv = jnp.where(lane == 2, by1, rv)
            rv = jnp.where(lane == 3, bx2, rv)
            rv = jnp.where(lane == 4, by2, rv)
            plsc.store_scatter(outflat, [k * 5 + lane], rv * keptf,
                               mask=row_mask)

            @pl.when(kept)
            def _():
                for j in range(NV):
                    sbase = j * 16
                    x1v = x1s[pl.ds(sbase, 16)]
                    y1v = y1s[pl.ds(sbase, 16)]
                    x2v = x2s[pl.ds(sbase, 16)]
                    y2v = y2s[pl.ds(sbase, 16)]
                    av = areas[pl.ds(sbase, 16)]
                    iw = jnp.maximum(jnp.minimum(bx2, x2v) - jnp.maximum(bx1, x1v), 0.0)
                    ih = jnp.maximum(jnp.minimum(by2, y2v) - jnp.maximum(by1, y1v), 0.0)
                    inter = iw * ih
                    iou = inter / (barea + av - inter + 1e-12)
                    sv_ = supp[pl.ds(sbase, 16)]
                    supp[pl.ds(sbase, 16)] = jnp.where(iou > NMS_T, 1.0, sv_)
            return carry

        lax.fori_loop(0, TOPK, extract, jnp.int32(0))
        pltpu.sync_copy(outflat.at[pl.ds(0, TOPK * 5)], out_hbm.at[b, c])

    @pl.when(half == 0)
    def _():
        def zblk(j, _):
            outflat[pl.ds(j * 16, 16)] = zero16
            return _
        lax.fori_loop(0, OUTF // 16, zblk, jnp.int32(0))
        pltpu.sync_copy(outflat.at[pl.ds(0, TOPK * 5)], out_hbm.at[b, 0])

    first = 1 + half * 10

    def cls_loop(i, _):
        th = thv[pl.ds(first + i, 16)][0]
        do_class(first + i, th)
        return _

    lax.fori_loop(0, 10, cls_loop, jnp.int32(0))


def _nms_sc(sc, bx, th, B):
    mesh = plsc.VectorSubcoreMesh(core_axis_name="c", subcore_axis_name="s")
    kern = functools.partial(
        pl.kernel,
        mesh=mesh,
        out_type=jax.ShapeDtypeStruct((B, NC, TOPK * 5), jnp.float32),
        compiler_params=pltpu.CompilerParams(
            needs_layout_passes=False, use_tc_tiling_on_sc=False),
        scratch_types=[
            pltpu.VMEM((NP,), jnp.float32),
            pltpu.VMEM((NP,), jnp.float32),
            pltpu.VMEM((NP,), jnp.float32),
            pltpu.VMEM((NP,), jnp.float32),
            pltpu.VMEM((NP,), jnp.float32),
            pltpu.VMEM((NCP,), jnp.float32),
            pltpu.VMEM((CAPX,), jnp.float32),
            pltpu.VMEM((CAPX,), jnp.int32),
            pltpu.VMEM((CAPX,), jnp.float32),
            pltpu.VMEM((CAPX,), jnp.float32),
            pltpu.VMEM((CAPX,), jnp.float32),
            pltpu.VMEM((CAPX,), jnp.float32),
            pltpu.VMEM((CAPX,), jnp.float32),
            pltpu.VMEM((CAPX,), jnp.float32),
            pltpu.VMEM((16,), jnp.float32),
            pltpu.VMEM((OUTF,), jnp.float32),
        ],
    )(_nms_body)
    return kern(sc, bx, th)


def kernel(loc_data, conf_data, dbox_list):
    B = loc_data.shape[0]
    sc, bx, th = _preprocess(loc_data, conf_data, dbox_list)
    out = _nms_sc(sc, bx, th, B)
    return out.reshape(B, NC, TOPK, 5)


# speedup vs baseline: 1.7915x; 1.2367x over previous
import functools

import jax
import jax.numpy as jnp
from jax import lax
from jax.experimental import pallas as pl
from jax.experimental.pallas import tpu as pltpu
from jax.experimental.pallas import tpu_sc as plsc

CONF = 0.01
TOPK = 200
NMS_T = 0.45
N = 8732
NP = 8736
NC = 21
NCP = 40
CAP = 208
NV = CAP // 16
CAPW = 224
CAPX = 256
NBLK = NP // 16
OUTF = 1024


def _pre_body(loc_ref, conf_ref, dbox_ref, sc_ref, bx_ref, th_ref):
    ct = conf_ref[0].T
    m = jnp.max(ct, axis=0, keepdims=True)
    e = jnp.exp(ct - m)
    z = e[0:1]
    for j in range(1, NC):
        z = z + e[j:j + 1]
    p = e / z
    st = jnp.where(p > CONF, p, 0.0)
    sc_ref[0] = jnp.concatenate([st, jnp.zeros((NC, NP - N), jnp.float32)], axis=1)

    loc = loc_ref[0]
    dbox = dbox_ref[...]
    cxcy = dbox[:, :2] + loc[:, :2] * 0.1 * dbox[:, :2]
    wh = dbox[:, 2:] * jnp.exp(loc[:, 2:] * 0.2)
    xy1 = cxcy - wh / 2.0
    xy2 = xy1 + wh
    bt = jnp.concatenate([xy1, xy2], axis=1).T
    bx_ref[0] = jnp.concatenate([bt, jnp.zeros((4, NP - N), jnp.float32)], axis=1)

    bits = lax.bitcast_convert_type(sc_ref[0], jnp.int32)

    def body(_, carry):
        lo, hi = carry
        mid = (lo + hi) // 2
        cnt = jnp.sum((bits > mid).astype(jnp.float32), axis=1, keepdims=True)
        pred = cnt >= float(TOPK)
        return jnp.where(pred, mid + 1, lo), jnp.where(pred, hi, mid)

    LOB = 0x3C23D70A
    lo0 = jnp.full((NC, 1), LOB, jnp.int32)
    hi0 = jnp.full((NC, 1), 0x3F800000, jnp.int32)
    _, hi = lax.fori_loop(0, 26, body, (lo0, hi0))
    thf = jnp.where(hi == LOB, 0.0, lax.bitcast_convert_type(hi, jnp.float32))
    th_ref[0] = jnp.concatenate(
        [thf.T, jnp.full((1, NCP - NC), 2.0, jnp.float32)], axis=1)


def _preprocess(loc_data, conf_data, dbox_list):
    B = loc_data.shape[0]
    return pl.pallas_call(
        _pre_body,
        grid=(B,),
        in_specs=[
            pl.BlockSpec((1, N, 4), lambda b: (b, 0, 0)),
            pl.BlockSpec((1, N, NC), lambda b: (b, 0, 0)),
            pl.BlockSpec((N, 4), lambda b: (0, 0)),
        ],
        out_specs=[
            pl.BlockSpec((1, NC, NP), lambda b: (b, 0, 0)),
            pl.BlockSpec((1, 4, NP), lambda b: (b, 0, 0)),
            pl.BlockSpec((1, 1, NCP), lambda b: (b, 0, 0)),
        ],
        out_shape=[
            jax.ShapeDtypeStruct((B, NC, NP), jnp.float32),
            jax.ShapeDtypeStruct((B, 4, NP), jnp.float32),
            jax.ShapeDtypeStruct((B, 1, NCP), jnp.float32),
        ],
        compiler_params=pltpu.CompilerParams(
            dimension_semantics=("parallel",)),
    )(loc_data, conf_data, dbox_list)


def _nms_body(sc_hbm, bx_hbm, th_hbm, out_hbm,
              bxp0, bxp1, bxp2, bxp3, srow, thv,
              ss, idxs, x1s, y1s, x2s, y2s, areas, supp, pvm, outflat):
    wid = lax.axis_index("s") * 2 + lax.axis_index("c")
    b = wid // 4
    quarter = wid % 4

    pltpu.sync_copy(bx_hbm.at[b, 0], bxp0)
    pltpu.sync_copy(bx_hbm.at[b, 1], bxp1)
    pltpu.sync_copy(bx_hbm.at[b, 2], bxp2)
    pltpu.sync_copy(bx_hbm.at[b, 3], bxp3)
    pltpu.sync_copy(th_hbm.at[b, 0], thv)

    lane = lax.iota(jnp.int32, 16)
    zero16 = jnp.zeros((16,), jnp.float32)
    row_mask = lane < 5

    def do_class(c, th):
        pltpu.sync_copy(sc_hbm.at[b, c], srow)

        for j in range(NV + 1):
            ss[pl.ds(j * 16, 16)] = jnp.full((16,), -1.0, jnp.float32)
            idxs[pl.ds(j * 16, 16)] = jnp.zeros((16,), jnp.int32)

        def comp_blk(h, cnt):
            for u in range(2):
                base = (h * 2 + u) * 16
                v = srow[pl.ds(base, 16)]
                msk = (v >= th) & (v > 0.0)
                woff = jnp.minimum(cnt, CAPW - 16)
                plsc.store_compressed(ss.at[pl.ds(woff, 16)], v, mask=msk)
                plsc.store_compressed(idxs.at[pl.ds(woff, 16)],
                                      base + lane, mask=msk)
                cnt = cnt + plsc.all_reduce_population_count(msk)[0]
            return cnt

        cnt = lax.fori_loop(0, NBLK // 2, comp_blk, jnp.int32(0))
        cnt = jnp.minimum(cnt, CAP)
        ss[pl.ds(cnt, 16)] = jnp.full((16,), -1.0, jnp.float32)
        idxs[pl.ds(cnt, 16)] = jnp.zeros((16,), jnp.int32)

        pvmv = jnp.full((16,), -1.0, jnp.float32)
        for j in range(NV):
            base = j * 16
            iv = idxs[pl.ds(base, 16)]
            x1v = plsc.load_gather(bxp0, [iv])
            y1v = plsc.load_gather(bxp1, [iv])
            x2v = plsc.load_gather(bxp2, [iv])
            y2v = plsc.load_gather(bxp3, [iv])
            x1s[pl.ds(base, 16)] = x1v
            y1s[pl.ds(base, 16)] = y1v
            x2s[pl.ds(base, 16)] = x2v
            y2s[pl.ds(base, 16)] = y2v
            areas[pl.ds(base, 16)] = (x2v - x1v) * (y2v - y1v)
            supp[pl.ds(base, 16)] = zero16
            pvmv = jnp.where(lane == j, jnp.max(ss[pl.ds(base, 16)]), pvmv)
        pvm[...] = pvmv

        def extract(k, carry):
            pv = pvm[...]
            gm = jnp.max(pv)
            v0 = jnp.minimum(plsc.all_reduce_ffs(pv == gm)[0], jnp.int32(15))
            base = v0 * 16
            sv = ss[pl.ds(base, 16)]
            l = jnp.minimum(plsc.all_reduce_ffs(sv == gm)[0], jnp.int32(15))
            sv2 = jnp.where(lane == l, -1.0, sv)
            ss[pl.ds(base, 16)] = sv2
            pvm[...] = jnp.where(lane == v0, jnp.max(sv2), pv)

            slot = base + l
            sup = supp[pl.ds(slot, 16)][0]
            bx1 = x1s[pl.ds(slot, 16)][0]
            by1 = y1s[pl.ds(slot, 16)][0]
            bx2 = x2s[pl.ds(slot, 16)][0]
            by2 = y2s[pl.ds(slot, 16)][0]
            barea = areas[pl.ds(slot, 16)][0]
            kept = (gm > 0.0) & (sup == 0.0)
            keptf = jnp.where(kept, 1.0, 0.0)

            rv = jnp.where(lane == 0, gm, zero16)
            rv = jnp.where(lane == 1, bx1, rv)
            rv = jnp.where(lane == 2, by1, rv)
            rv = jnp.where(lane == 3, bx2, rv)
            rv = jnp.where(lane == 4, by2, rv)
            plsc.store_scatter(outflat, [k * 5 + lane], rv * keptf,
                               mask=row_mask)

            @pl.when(kept)
            def _():
                for j in range(NV):
                    sbase = j * 16
                    x1v = x1s[pl.ds(sbase, 16)]
                    y1v = y1s[pl.ds(sbase, 16)]
                    x2v = x2s[pl.ds(sbase, 16)]
                    y2v = y2s[pl.ds(sbase, 16)]
                    av = areas[pl.ds(sbase, 16)]
                    iw = jnp.maximum(jnp.minimum(bx2, x2v) - jnp.maximum(bx1, x1v), 0.0)
                    ih = jnp.maximum(jnp.minimum(by2, y2v) - jnp.maximum(by1, y1v), 0.0)
                    inter = iw * ih
                    iou = inter / (barea + av - inter + 1e-12)
                    sv_ = supp[pl.ds(sbase, 16)]
                    supp[pl.ds(sbase, 16)] = jnp.where(iou > NMS_T, 1.0, sv_)
            return carry

        lax.fori_loop(0, TOPK, extract, jnp.int32(0))
        pltpu.sync_copy(outflat.at[pl.ds(0, TOPK * 5)], out_hbm.at[b, c])

    @pl.when(quarter == 0)
    def _():
        def zblk(j, _):
            outflat[pl.ds(j * 16, 16)] = zero16
            return _
        lax.fori_loop(0, OUTF // 16, zblk, jnp.int32(0))
        pltpu.sync_copy(outflat.at[pl.ds(0, TOPK * 5)], out_hbm.at[b, 0])

    first = 1 + quarter * 5

    def cls_loop(i, _):
        th = thv[pl.ds(first + i, 16)][0]
        do_class(first + i, th)
        return _

    lax.fori_loop(0, 5, cls_loop, jnp.int32(0))


def _nms_sc(sc, bx, th, B):
    mesh = plsc.VectorSubcoreMesh(core_axis_name="c", subcore_axis_name="s")
    kern = functools.partial(
        pl.kernel,
        mesh=mesh,
        out_type=jax.ShapeDtypeStruct((B, NC, TOPK * 5), jnp.float32),
        compiler_params=pltpu.CompilerParams(
            needs_layout_passes=False, use_tc_tiling_on_sc=False),
        scratch_types=[
            pltpu.VMEM((NP,), jnp.float32),
            pltpu.VMEM((NP,), jnp.float32),
            pltpu.VMEM((NP,), jnp.float32),
            pltpu.VMEM((NP,), jnp.float32),
            pltpu.VMEM((NP,), jnp.float32),
            pltpu.VMEM((NCP,), jnp.float32),
            pltpu.VMEM((CAPX,), jnp.float32),
            pltpu.VMEM((CAPX,), jnp.int32),
            pltpu.VMEM((CAPX,), jnp.float32),
            pltpu.VMEM((CAPX,), jnp.float32),
            pltpu.VMEM((CAPX,), jnp.float32),
            pltpu.VMEM((CAPX,), jnp.float32),
            pltpu.VMEM((CAPX,), jnp.float32),
            pltpu.VMEM((CAPX,), jnp.float32),
            pltpu.VMEM((16,), jnp.float32),
            pltpu.VMEM((OUTF,), jnp.float32),
        ],
    )(_nms_body)
    return kern(sc, bx, th)


def kernel(loc_data, conf_data, dbox_list):
    B = loc_data.shape[0]
    G = 8
    outs = []
    for g0 in range(0, B, G):
        sc, bx, th = _preprocess(loc_data[g0:g0 + G], conf_data[g0:g0 + G],
                                 dbox_list)
        outs.append(_nms_sc(sc, bx, th, G))
    return jnp.concatenate(outs, axis=0).reshape(B, NC, TOPK, 5)


# speedup vs baseline: 1.7950x; 1.0020x over previous
import functools

import jax
import jax.numpy as jnp
from jax import lax
from jax.experimental import pallas as pl
from jax.experimental.pallas import tpu as pltpu
from jax.experimental.pallas import tpu_sc as plsc

CONF = 0.01
TOPK = 200
NMS_T = 0.45
N = 8732
NP = 8736
NC = 21
NCP = 40
CAP = 208
NV = CAP // 16
CAPW = 224
CAPX = 256
NBLK = NP // 16
OUTF = 1024


def _pre_body(loc_ref, conf_ref, dbox_ref, sc_ref, bx_ref, th_ref):
    ct = conf_ref[0].T
    m = jnp.max(ct, axis=0, keepdims=True)
    e = jnp.exp(ct - m)
    z = e[0:1]
    for j in range(1, NC):
        z = z + e[j:j + 1]
    p = e / z
    st = jnp.where(p > CONF, p, 0.0)
    sc_ref[0] = jnp.concatenate([st, jnp.zeros((NC, NP - N), jnp.float32)], axis=1)

    loc = loc_ref[0]
    dbox = dbox_ref[...]
    cxcy = dbox[:, :2] + loc[:, :2] * 0.1 * dbox[:, :2]
    wh = dbox[:, 2:] * jnp.exp(loc[:, 2:] * 0.2)
    xy1 = cxcy - wh / 2.0
    xy2 = xy1 + wh
    bt = jnp.concatenate([xy1, xy2], axis=1).T
    bx_ref[0] = jnp.concatenate([bt, jnp.zeros((4, NP - N), jnp.float32)], axis=1)

    bits = lax.bitcast_convert_type(sc_ref[0], jnp.int32)

    def body(_, carry):
        lo, hi = carry
        mid = (lo + hi) // 2
        cnt = jnp.sum((bits > mid).astype(jnp.float32), axis=1, keepdims=True)
        pred = cnt >= float(TOPK)
        return jnp.where(pred, mid + 1, lo), jnp.where(pred, hi, mid)

    LOB = 0x3C23D70A
    lo0 = jnp.full((NC, 1), LOB, jnp.int32)
    hi0 = jnp.full((NC, 1), 0x3F800000, jnp.int32)
    _, hi = lax.fori_loop(0, 26, body, (lo0, hi0))
    thf = jnp.where(hi == LOB, 0.0, lax.bitcast_convert_type(hi, jnp.float32))
    th_ref[0] = jnp.concatenate(
        [thf.T, jnp.full((1, NCP - NC), 2.0, jnp.float32)], axis=1)


def _preprocess(loc_data, conf_data, dbox_list):
    B = loc_data.shape[0]
    return pl.pallas_call(
        _pre_body,
        grid=(B,),
        in_specs=[
            pl.BlockSpec((1, N, 4), lambda b: (b, 0, 0)),
            pl.BlockSpec((1, N, NC), lambda b: (b, 0, 0)),
            pl.BlockSpec((N, 4), lambda b: (0, 0)),
        ],
        out_specs=[
            pl.BlockSpec((1, NC, NP), lambda b: (b, 0, 0)),
            pl.BlockSpec((1, 4, NP), lambda b: (b, 0, 0)),
            pl.BlockSpec((1, 1, NCP), lambda b: (b, 0, 0)),
        ],
        out_shape=[
            jax.ShapeDtypeStruct((B, NC, NP), jnp.float32),
            jax.ShapeDtypeStruct((B, 4, NP), jnp.float32),
            jax.ShapeDtypeStruct((B, 1, NCP), jnp.float32),
        ],
        compiler_params=pltpu.CompilerParams(
            dimension_semantics=("parallel",)),
    )(loc_data, conf_data, dbox_list)


def _nms_body(sc_hbm, bx_hbm, th_hbm, out_hbm,
              bxp0, bxp1, bxp2, bxp3, srow0, srow1, thv,
              ss, idxs, x1s, y1s, x2s, y2s, areas, supp, pvm,
              outflat0, outflat1, sin0, sin1, sout0, sout1):
    wid = lax.axis_index("s") * 2 + lax.axis_index("c")
    b = wid // 4
    quarter = wid % 4

    pltpu.sync_copy(bx_hbm.at[b, 0], bxp0)
    pltpu.sync_copy(bx_hbm.at[b, 1], bxp1)
    pltpu.sync_copy(bx_hbm.at[b, 2], bxp2)
    pltpu.sync_copy(bx_hbm.at[b, 3], bxp3)
    pltpu.sync_copy(th_hbm.at[b, 0], thv)

    lane = lax.iota(jnp.int32, 16)
    zero16 = jnp.zeros((16,), jnp.float32)
    row_mask = lane < 5

    def do_class(c, th, srow, outflat):

        for j in range(NV + 1):
            ss[pl.ds(j * 16, 16)] = jnp.full((16,), -1.0, jnp.float32)
            idxs[pl.ds(j * 16, 16)] = jnp.zeros((16,), jnp.int32)

        def comp_blk(h, cnt):
            for u in range(2):
                base = (h * 2 + u) * 16
                v = srow[pl.ds(base, 16)]
                msk = (v >= th) & (v > 0.0)
                woff = jnp.minimum(cnt, CAPW - 16)
                plsc.store_compressed(ss.at[pl.ds(woff, 16)], v, mask=msk)
                plsc.store_compressed(idxs.at[pl.ds(woff, 16)],
                                      base + lane, mask=msk)
                cnt = cnt + plsc.all_reduce_population_count(msk)[0]
            return cnt

        cnt = lax.fori_loop(0, NBLK // 2, comp_blk, jnp.int32(0))
        cnt = jnp.minimum(cnt, CAP)
        ss[pl.ds(cnt, 16)] = jnp.full((16,), -1.0, jnp.float32)
        idxs[pl.ds(cnt, 16)] = jnp.zeros((16,), jnp.int32)

        pvmv = jnp.full((16,), -1.0, jnp.float32)
        for j in range(NV):
            base = j * 16
            iv = idxs[pl.ds(base, 16)]
            x1v = plsc.load_gather(bxp0, [iv])
            y1v = plsc.load_gather(bxp1, [iv])
            x2v = plsc.load_gather(bxp2, [iv])
            y2v = plsc.load_gather(bxp3, [iv])
            x1s[pl.ds(base, 16)] = x1v
            y1s[pl.ds(base, 16)] = y1v
            x2s[pl.ds(base, 16)] = x2v
            y2s[pl.ds(base, 16)] = y2v
            areas[pl.ds(base, 16)] = (x2v - x1v) * (y2v - y1v)
            supp[pl.ds(base, 16)] = zero16
            pvmv = jnp.where(lane == j, jnp.max(ss[pl.ds(base, 16)]), pvmv)
        pvm[...] = pvmv

        def extract(k, carry):
            pv = pvm[...]
            gm = jnp.max(pv)
            v0 = jnp.minimum(plsc.all_reduce_ffs(pv == gm)[0], jnp.int32(15))
            base = v0 * 16
            sv = ss[pl.ds(base, 16)]
            l = jnp.minimum(plsc.all_reduce_ffs(sv == gm)[0], jnp.int32(15))
            sv2 = jnp.where(lane == l, -1.0, sv)
            ss[pl.ds(base, 16)] = sv2
            pvm[...] = jnp.where(lane == v0, jnp.max(sv2), pv)

            slot = base + l
            sup = supp[pl.ds(slot, 16)][0]
            bx1 = x1s[pl.ds(slot, 16)][0]
            by1 = y1s[pl.ds(slot, 16)][0]
            bx2 = x2s[pl.ds(slot, 16)][0]
            by2 = y2s[pl.ds(slot, 16)][0]
            barea = areas[pl.ds(slot, 16)][0]
            kept = (gm > 0.0) & (sup == 0.0)
            keptf = jnp.where(kept, 1.0, 0.0)

            rv = jnp.where(lane == 0, gm, zero16)
            rv = jnp.where(lane == 1, bx1, rv)
            rv = jnp.where(lane == 2, by1, rv)
            rv = jnp.where(lane == 3, bx2, rv)
            rv = jnp.where(lane == 4, by2, rv)
            plsc.store_scatter(outflat, [k * 5 + lane], rv * keptf,
                               mask=row_mask)

            @pl.when(kept)
            def _():
                for j in range(NV):
                    sbase = j * 16
                    x1v = x1s[pl.ds(sbase, 16)]
                    y1v = y1s[pl.ds(sbase, 16)]
                    x2v = x2s[pl.ds(sbase, 16)]
                    y2v = y2s[pl.ds(sbase, 16)]
                    av = areas[pl.ds(sbase, 16)]
                    iw = jnp.maximum(jnp.minimum(bx2, x2v) - jnp.maximum(bx1, x1v), 0.0)
                    ih = jnp.maximum(jnp.minimum(by2, y2v) - jnp.maximum(by1, y1v), 0.0)
                    inter = iw * ih
                    iou = inter / (barea + av - inter + 1e-12)
                    sv_ = supp[pl.ds(sbase, 16)]
                    supp[pl.ds(sbase, 16)] = jnp.where(iou > NMS_T, 1.0, sv_)
            return carry

        lax.fori_loop(0, TOPK, extract, jnp.int32(0))

    @pl.when(quarter == 0)
    def _():
        def zblk(j, _):
            outflat0[pl.ds(j * 16, 16)] = zero16
            return _
        lax.fori_loop(0, OUTF // 16, zblk, jnp.int32(0))
        pltpu.sync_copy(outflat0.at[pl.ds(0, TOPK * 5)], out_hbm.at[b, 0])

    first = 1 + quarter * 5
    srows = (srow0, srow1)
    outs = (outflat0, outflat1)
    sins = (sin0, sin1)
    souts = (sout0, sout1)

    in_h = [None, None]
    out_h = [None, None]
    in_h[0] = pltpu.async_copy(sc_hbm.at[b, first], srow0, sin0)
    in_h[1] = pltpu.async_copy(sc_hbm.at[b, first + 1], srow1, sin1)
    for i in range(5):
        p = i % 2
        th = thv[pl.ds(first + i, 16)][0]
        in_h[p].wait()
        if out_h[p] is not None:
            out_h[p].wait()
        do_class(first + i, th, srows[p], outs[p])
        if i + 2 < 5:
            in_h[p] = pltpu.async_copy(sc_hbm.at[b, first + i + 2],
                                       srows[p], sins[p])
        out_h[p] = pltpu.async_copy(outs[p].at[pl.ds(0, TOPK * 5)],
                                    out_hbm.at[b, first + i], souts[p])
    out_h[0].wait()
    out_h[1].wait()


def _nms_sc(sc, bx, th, B):
    mesh = plsc.VectorSubcoreMesh(core_axis_name="c", subcore_axis_name="s")
    kern = functools.partial(
        pl.kernel,
        mesh=mesh,
        out_type=jax.ShapeDtypeStruct((B, NC, TOPK * 5), jnp.float32),
        compiler_params=pltpu.CompilerParams(
            needs_layout_passes=False, use_tc_tiling_on_sc=False),
        scratch_types=[
            pltpu.VMEM((NP,), jnp.float32),
            pltpu.VMEM((NP,), jnp.float32),
            pltpu.VMEM((NP,), jnp.float32),
            pltpu.VMEM((NP,), jnp.float32),
            pltpu.VMEM((NP,), jnp.float32),
            pltpu.VMEM((NP,), jnp.float32),
            pltpu.VMEM((NCP,), jnp.float32),
            pltpu.VMEM((CAPX,), jnp.float32),
            pltpu.VMEM((CAPX,), jnp.int32),
            pltpu.VMEM((CAPX,), jnp.float32),
            pltpu.VMEM((CAPX,), jnp.float32),
            pltpu.VMEM((CAPX,), jnp.float32),
            pltpu.VMEM((CAPX,), jnp.float32),
            pltpu.VMEM((CAPX,), jnp.float32),
            pltpu.VMEM((CAPX,), jnp.float32),
            pltpu.VMEM((16,), jnp.float32),
            pltpu.VMEM((OUTF,), jnp.float32),
            pltpu.VMEM((OUTF,), jnp.float32),
            pltpu.SemaphoreType.DMA,
            pltpu.SemaphoreType.DMA,
            pltpu.SemaphoreType.DMA,
            pltpu.SemaphoreType.DMA,
        ],
    )(_nms_body)
    return kern(sc, bx, th)


def kernel(loc_data, conf_data, dbox_list):
    B = loc_data.shape[0]
    G = 8
    outs = []
    for g0 in range(0, B, G):
        sc, bx, th = _preprocess(loc_data[g0:g0 + G], conf_data[g0:g0 + G],
                                 dbox_list)
        outs.append(_nms_sc(sc, bx, th, G))
    return jnp.concatenate(outs, axis=0).reshape(B, NC, TOPK, 5)


# speedup vs baseline: 1.8063x; 1.0063x over previous
import functools

import jax
import jax.numpy as jnp
from jax import lax
from jax.experimental import pallas as pl
from jax.experimental.pallas import tpu as pltpu
from jax.experimental.pallas import tpu_sc as plsc

CONF = 0.01
TOPK = 200
NMS_T = 0.45
N = 8732
NP = 8736
NC = 21
NCP = 40
CAP = 208
NV = CAP // 16
CAPW = 224
CAPX = 256
NBLK = NP // 16
OUTF = 1024


def _pre_body(loc_ref, conf_ref, dbox_ref, sc_ref, bx_ref, th_ref):
    ct = conf_ref[0].T
    m = jnp.max(ct, axis=0, keepdims=True)
    e = jnp.exp(ct - m)
    z = e[0:1]
    for j in range(1, NC):
        z = z + e[j:j + 1]
    p = e / z
    st = jnp.where(p > CONF, p, 0.0)
    sc_ref[0] = jnp.concatenate([st, jnp.zeros((NC, NP - N), jnp.float32)], axis=1)

    loc = loc_ref[0]
    dbox = dbox_ref[...]
    cxcy = dbox[:, :2] + loc[:, :2] * 0.1 * dbox[:, :2]
    wh = dbox[:, 2:] * jnp.exp(loc[:, 2:] * 0.2)
    xy1 = cxcy - wh / 2.0
    xy2 = xy1 + wh
    bt = jnp.concatenate([xy1, xy2], axis=1).T
    bx_ref[0] = jnp.concatenate([bt, jnp.zeros((4, NP - N), jnp.float32)], axis=1)

    bits = lax.bitcast_convert_type(sc_ref[0], jnp.int32)

    def body(_, carry):
        lo, hi = carry
        mid = (lo + hi) // 2
        cnt = jnp.sum((bits > mid).astype(jnp.float32), axis=1, keepdims=True)
        pred = cnt >= float(TOPK)
        return jnp.where(pred, mid + 1, lo), jnp.where(pred, hi, mid)

    LOB = 0x3C23D70A
    lo0 = jnp.full((NC, 1), LOB, jnp.int32)
    hi0 = jnp.full((NC, 1), 0x3F800000, jnp.int32)
    _, hi = lax.fori_loop(0, 26, body, (lo0, hi0))
    thf = jnp.where(hi == LOB, 0.0, lax.bitcast_convert_type(hi, jnp.float32))
    th_ref[0] = jnp.concatenate(
        [thf.T, jnp.full((1, NCP - NC), 2.0, jnp.float32)], axis=1)


def _preprocess(loc_data, conf_data, dbox_list):
    B = loc_data.shape[0]
    return pl.pallas_call(
        _pre_body,
        grid=(B,),
        in_specs=[
            pl.BlockSpec((1, N, 4), lambda b: (b, 0, 0)),
            pl.BlockSpec((1, N, NC), lambda b: (b, 0, 0)),
            pl.BlockSpec((N, 4), lambda b: (0, 0)),
        ],
        out_specs=[
            pl.BlockSpec((1, NC, NP), lambda b: (b, 0, 0)),
            pl.BlockSpec((1, 4, NP), lambda b: (b, 0, 0)),
            pl.BlockSpec((1, 1, NCP), lambda b: (b, 0, 0)),
        ],
        out_shape=[
            jax.ShapeDtypeStruct((B, NC, NP), jnp.float32),
            jax.ShapeDtypeStruct((B, 4, NP), jnp.float32),
            jax.ShapeDtypeStruct((B, 1, NCP), jnp.float32),
        ],
        compiler_params=pltpu.CompilerParams(
            dimension_semantics=("parallel",)),
    )(loc_data, conf_data, dbox_list)


def _nms_body(sc_hbm, bx_hbm, th_hbm, out_hbm,
              bxp0, bxp1, bxp2, bxp3, srow0, srow1, thv,
              ss, idxs, x1s, y1s, x2s, y2s, areas, supp, pvm,
              outflat0, outflat1, sin0, sin1, sout0, sout1):
    wid = lax.axis_index("s") * 2 + lax.axis_index("c")
    b = wid // 4
    quarter = wid % 4

    pltpu.sync_copy(bx_hbm.at[b, 0], bxp0)
    pltpu.sync_copy(bx_hbm.at[b, 1], bxp1)
    pltpu.sync_copy(bx_hbm.at[b, 2], bxp2)
    pltpu.sync_copy(bx_hbm.at[b, 3], bxp3)
    pltpu.sync_copy(th_hbm.at[b, 0], thv)

    lane = lax.iota(jnp.int32, 16)
    zero16 = jnp.zeros((16,), jnp.float32)
    row_mask = lane < 5

    def do_class(c, th, srow, outflat):

        for j in range(NV + 1):
            ss[pl.ds(j * 16, 16)] = jnp.full((16,), -1.0, jnp.float32)
            idxs[pl.ds(j * 16, 16)] = jnp.zeros((16,), jnp.int32)

        def comp_blk(h, cnt):
            for u in range(2):
                base = (h * 2 + u) * 16
                v = srow[pl.ds(base, 16)]
                msk = (v >= th) & (v > 0.0)
                woff = jnp.minimum(cnt, CAPW - 16)
                plsc.store_compressed(ss.at[pl.ds(woff, 16)], v, mask=msk)
                plsc.store_compressed(idxs.at[pl.ds(woff, 16)],
                                      base + lane, mask=msk)
                cnt = cnt + plsc.all_reduce_population_count(msk)[0]
            return cnt

        cnt = lax.fori_loop(0, NBLK // 2, comp_blk, jnp.int32(0))
        cnt = jnp.minimum(cnt, CAP)
        ss[pl.ds(cnt, 16)] = jnp.full((16,), -1.0, jnp.float32)
        idxs[pl.ds(cnt, 16)] = jnp.zeros((16,), jnp.int32)

        pvmv = jnp.full((16,), -1.0, jnp.float32)
        for j in range(NV):
            base = j * 16
            iv = idxs[pl.ds(base, 16)]
            x1v = plsc.load_gather(bxp0, [iv])
            y1v = plsc.load_gather(bxp1, [iv])
            x2v = plsc.load_gather(bxp2, [iv])
            y2v = plsc.load_gather(bxp3, [iv])
            x1s[pl.ds(base, 16)] = x1v
            y1s[pl.ds(base, 16)] = y1v
            x2s[pl.ds(base, 16)] = x2v
            y2s[pl.ds(base, 16)] = y2v
            areas[pl.ds(base, 16)] = (x2v - x1v) * (y2v - y1v)
            supp[pl.ds(base, 16)] = zero16
            pvmv = jnp.where(lane == j, jnp.max(ss[pl.ds(base, 16)]), pvmv)
        pvm[...] = pvmv

        def extract(k, carry):
            pv = pvm[...]
            gm = jnp.max(pv)
            v0 = jnp.minimum(plsc.all_reduce_ffs(pv == gm)[0], jnp.int32(15))
            base = v0 * 16
            sv = ss[pl.ds(base, 16)]
            l = jnp.minimum(plsc.all_reduce_ffs(sv == gm)[0], jnp.int32(15))
            sv2 = jnp.where(lane == l, -1.0, sv)
            ss[pl.ds(base, 16)] = sv2
            pvm[...] = jnp.where(lane == v0, jnp.max(sv2), pv)

            slot = base + l
            sup = supp[pl.ds(slot, 16)][0]
            bx1 = x1s[pl.ds(slot, 16)][0]
            by1 = y1s[pl.ds(slot, 16)][0]
            bx2 = x2s[pl.ds(slot, 16)][0]
            by2 = y2s[pl.ds(slot, 16)][0]
            barea = areas[pl.ds(slot, 16)][0]
            kept = (gm > 0.0) & (sup == 0.0)
            keptf = jnp.where(kept, 1.0, 0.0)

            rv = jnp.where(lane == 0, gm, zero16)
            rv = jnp.where(lane == 1, bx1, rv)
            rv = jnp.where(lane == 2, by1, rv)
            rv = jnp.where(lane == 3, bx2, rv)
            rv = jnp.where(lane == 4, by2, rv)
            plsc.store_scatter(outflat, [k * 5 + lane], rv * keptf,
                               mask=row_mask)

            @pl.when(kept)
            def _():
                for j in range(NV):
                    sbase = j * 16
                    x1v = x1s[pl.ds(sbase, 16)]
                    y1v = y1s[pl.ds(sbase, 16)]
                    x2v = x2s[pl.ds(sbase, 16)]
                    y2v = y2s[pl.ds(sbase, 16)]
                    av = areas[pl.ds(sbase, 16)]
                    iw = jnp.maximum(jnp.minimum(bx2, x2v) - jnp.maximum(bx1, x1v), 0.0)
                    ih = jnp.maximum(jnp.minimum(by2, y2v) - jnp.maximum(by1, y1v), 0.0)
                    inter = iw * ih
                    rhs = NMS_T * (barea + av - inter + 1e-12)
                    sv_ = supp[pl.ds(sbase, 16)]
                    supp[pl.ds(sbase, 16)] = jnp.where(inter > rhs, 1.0, sv_)
            return carry

        lax.fori_loop(0, TOPK, extract, jnp.int32(0))

    @pl.when(quarter == 0)
    def _():
        def zblk(j, _):
            outflat0[pl.ds(j * 16, 16)] = zero16
            return _
        lax.fori_loop(0, OUTF // 16, zblk, jnp.int32(0))
        pltpu.sync_copy(outflat0.at[pl.ds(0, TOPK * 5)], out_hbm.at[b, 0])

    first = 1 + quarter * 5
    srows = (srow0, srow1)
    outs = (outflat0, outflat1)
    sins = (sin0, sin1)
    souts = (sout0, sout1)

    in_h = [None, None]
    out_h = [None, None]
    in_h[0] = pltpu.async_copy(sc_hbm.at[b, first], srow0, sin0)
    in_h[1] = pltpu.async_copy(sc_hbm.at[b, first + 1], srow1, sin1)
    for i in range(5):
        p = i % 2
        th = thv[pl.ds(first + i, 16)][0]
        in_h[p].wait()
        if out_h[p] is not None:
            out_h[p].wait()
        do_class(first + i, th, srows[p], outs[p])
        if i + 2 < 5:
            in_h[p] = pltpu.async_copy(sc_hbm.at[b, first + i + 2],
                                       srows[p], sins[p])
        out_h[p] = pltpu.async_copy(outs[p].at[pl.ds(0, TOPK * 5)],
                                    out_hbm.at[b, first + i], souts[p])
    out_h[0].wait()
    out_h[1].wait()


def _nms_sc(sc, bx, th, B):
    mesh = plsc.VectorSubcoreMesh(core_axis_name="c", subcore_axis_name="s")
    kern = functools.partial(
        pl.kernel,
        mesh=mesh,
        out_type=jax.ShapeDtypeStruct((B, NC, TOPK * 5), jnp.float32),
        compiler_params=pltpu.CompilerParams(
            needs_layout_passes=False, use_tc_tiling_on_sc=False),
        scratch_types=[
            pltpu.VMEM((NP,), jnp.float32),
            pltpu.VMEM((NP,), jnp.float32),
            pltpu.VMEM((NP,), jnp.float32),
            pltpu.VMEM((NP,), jnp.float32),
            pltpu.VMEM((NP,), jnp.float32),
            pltpu.VMEM((NP,), jnp.float32),
            pltpu.VMEM((NCP,), jnp.float32),
            pltpu.VMEM((CAPX,), jnp.float32),
            pltpu.VMEM((CAPX,), jnp.int32),
            pltpu.VMEM((CAPX,), jnp.float32),
            pltpu.VMEM((CAPX,), jnp.float32),
            pltpu.VMEM((CAPX,), jnp.float32),
            pltpu.VMEM((CAPX,), jnp.float32),
            pltpu.VMEM((CAPX,), jnp.float32),
            pltpu.VMEM((CAPX,), jnp.float32),
            pltpu.VMEM((16,), jnp.float32),
            pltpu.VMEM((OUTF,), jnp.float32),
            pltpu.VMEM((OUTF,), jnp.float32),
            pltpu.SemaphoreType.DMA,
            pltpu.SemaphoreType.DMA,
            pltpu.SemaphoreType.DMA,
            pltpu.SemaphoreType.DMA,
        ],
    )(_nms_body)
    return kern(sc, bx, th)


def kernel(loc_data, conf_data, dbox_list):
    B = loc_data.shape[0]
    G = 8
    outs = []
    for g0 in range(0, B, G):
        sc, bx, th = _preprocess(loc_data[g0:g0 + G], conf_data[g0:g0 + G],
                                 dbox_list)
        outs.append(_nms_sc(sc, bx, th, G))
    return jnp.concatenate(outs, axis=0).reshape(B, NC, TOPK, 5)
